# baseline (speedup 1.0000x reference)
import functools

import jax
import jax.numpy as jnp
from jax import lax
from jax.experimental import pallas as pl
from jax.experimental.pallas import tpu as pltpu, tpu_sc as plsc

_LANES = 16


def _build_sc_call(B, C, H, W, N):
    info = plsc.get_sparse_core_info()
    NC, NS = info.num_cores, info.num_subcores
    NW = NC * NS
    P = B * N
    PW = P // NW
    K = 128
    NCHUNK = PW // K
    WPB = NW // B
    CJ = C // _LANES

    mesh = plsc.VectorSubcoreMesh(core_axis_name="c", subcore_axis_name="s")

    @functools.partial(
        pl.kernel,
        mesh=mesh,
        out_type=jax.ShapeDtypeStruct((B, N, C), jnp.float32),
        scratch_types=[
            pltpu.VMEM((PW * 2 // 128, 128), jnp.int32),
            pltpu.VMEM((4 * K + _LANES,), jnp.float32),
            pltpu.VMEM((4 * K + _LANES,), jnp.float32),
            pltpu.VMEM((2 * K, 128), jnp.int32),
            pltpu.VMEM((2 * K, 128), jnp.int32),
            pltpu.VMEM((K, C), jnp.float32),
            pltpu.SemaphoreType.DMA,
            pltpu.SemaphoreType.DMA,
        ],
    )
    def sc_fn(tbl, idxh, wh, out, idx_all, w_a, w_b, rows_a, rows_b, outT, sem_a, sem_b):
        wid = lax.axis_index("s") * NC + lax.axis_index("c")
        bi = wid // WPB

        pltpu.sync_copy(idxh.at[pl.ds(wid * (PW * 2 // 128), PW * 2 // 128)], idx_all)

        def fire(n, rows, wbuf, sem):
            pltpu.async_copy(tbl.at[idx_all.at[2 * n]], rows.at[pl.ds(0, 128)], sem)
            pltpu.async_copy(tbl.at[idx_all.at[2 * n + 1]], rows.at[pl.ds(128, 128)], sem)
            pltpu.async_copy(wh.at[pl.ds(wid * PW * 4 + n * 4 * K, 4 * K)],
                             wbuf.at[pl.ds(0, 4 * K)], sem)

        def drain(rows, wbuf, sem):
            pltpu.make_async_copy(tbl.at[idx_all.at[0]], rows.at[pl.ds(0, 128)], sem).wait()
            pltpu.make_async_copy(tbl.at[idx_all.at[0]], rows.at[pl.ds(128, 128)], sem).wait()
            pltpu.make_async_copy(wh.at[pl.ds(0, 4 * K)], wbuf.at[pl.ds(0, 4 * K)], sem).wait()

        def compute(n, rows, wbuf):
            def pt_body(p, carry):
                wv = wbuf[pl.ds(4 * p, _LANES)]
                w0 = wv[0]
                w1 = wv[1]
                w2 = wv[2]
                w3 = wv[3]
                r = 2 * p
                hi = jnp.int32(-65536)

                v0s = [rows[r, pl.ds(_LANES * g, _LANES)] for g in range(CJ)]
                v1s = [rows[r + 1, pl.ds(_LANES * g, _LANES)] for g in range(CJ)]
                for g in range(CJ):
                    a = lax.bitcast_convert_type(v0s[g] << 16, jnp.float32)
                    b = lax.bitcast_convert_type(v0s[g] & hi, jnp.float32)
                    c = lax.bitcast_convert_type(v1s[g] << 16, jnp.float32)
                    d = lax.bitcast_convert_type(v1s[g] & hi, jnp.float32)
                    outT[p, pl.ds(_LANES * g, _LANES)] = (w0 * a + w1 * b) + (w2 * c + w3 * d)
                return carry

            lax.fori_loop(0, K, pt_body, 0, unroll=2)

        def emit(n):
            offb = (wid % WPB) * PW + n * K
            pltpu.sync_copy(outT, out.at[bi, pl.ds(offb, K), :])

        fire(0, rows_a, w_a, sem_a)

        def pair_body(t, carry):
            n0 = 2 * t
            drain(rows_a, w_a, sem_a)
            fire(n0 + 1, rows_b, w_b, sem_b)
            compute(n0, rows_a, w_a)
            emit(n0)
            drain(rows_b, w_b, sem_b)

            @pl.when(n0 + 2 < NCHUNK)
            def _():
                fire(n0 + 2, rows_a, w_a, sem_a)

            compute(n0 + 1, rows_b, w_b)
            emit(n0 + 1)
            return carry

        lax.fori_loop(0, NCHUNK // 2, pair_body, 0)

    return sc_fn


def _tc_pack_table(grid_feat):
    B, C, H, W = grid_feat.shape
    HB = 8

    def body(x_ref, o_ref):
        xb = x_ref[...].astype(jnp.bfloat16)
        u = lax.bitcast_convert_type(xb, jnp.uint16).astype(jnp.uint32)
        un = jnp.concatenate([u[:, :, 1:], u[:, :, -1:]], axis=2)
        w = lax.bitcast_convert_type(u | (un << 16), jnp.int32)
        o_ref[:, 0:C] = w.reshape(C, HB * W).T
        o_ref[:, C:128] = jnp.zeros((HB * W, 128 - C), jnp.int32)

    return pl.pallas_call(
        body,
        grid=(B, H // HB),
        in_specs=[pl.BlockSpec((None, C, HB, W), lambda b, i: (b, 0, i, 0))],
        out_specs=pl.BlockSpec((HB * W, 128), lambda b, i: (b * (H // HB) + i, 0)),
        out_shape=jax.ShapeDtypeStruct((B * H * W, 128), jnp.int32),
    )(grid_feat)


def _tc_transpose(x, out_shape):
    B, N, C = x.shape
    _, _, P0, P1 = out_shape
    TN = 8192
    YS = TN // P1

    def body(x_ref, o_ref):
        for y in range(YS):
            o_ref[:, y, :] = x_ref[y * P1:(y + 1) * P1, :].T

    return pl.pallas_call(
        body,
        grid=(B, N // TN),
        in_specs=[pl.BlockSpec((None, TN, C), lambda b, i: (b, i, 0))],
        out_specs=pl.BlockSpec((None, C, YS, P1), lambda b, i: (b, 0, i, 0)),
        out_shape=jax.ShapeDtypeStruct(out_shape, x.dtype),
    )(x)


def _point_tables(gsi_slice, B, C, H, W, N):
    P = B * N
    g = gsi_slice.reshape(P, 2)
    x = (g[:, 0] + 1.0) * 0.5 * (W - 1)
    y = (g[:, 1] + 1.0) * 0.5 * (H - 1)
    x0 = jnp.floor(x)
    y0 = jnp.floor(y)
    wx1 = x - x0
    wx0 = 1.0 - wx1
    wy1 = y - y0
    wy0 = 1.0 - wy1
    xi0 = x0.astype(jnp.int32)
    yi0 = y0.astype(jnp.int32)
    bx = jnp.clip(xi0, 0, W - 2)
    by = jnp.clip(yi0, 0, H - 2)
    wxa = wx0 * (bx == xi0) + wx1 * (bx == xi0 + 1)
    wxb = wx0 * (bx + 1 == xi0) + wx1 * (bx + 1 == xi0 + 1)
    wya = wy0 * (by == yi0) + wy1 * (by == yi0 + 1)
    wyb = wy0 * (by + 1 == yi0) + wy1 * (by + 1 == yi0 + 1)

    bidx = jnp.repeat(jnp.arange(B, dtype=jnp.int32), N)
    base = (bidx * H + by) * W + bx
    idx2 = jnp.stack([base, base + W], axis=1)
    w4 = jnp.stack([wya * wxa, wya * wxb, wyb * wxa, wyb * wxb], axis=1).reshape(P * 4)
    return idx2.reshape(P * 2 // 128, 128), w4


def kernel(grid_feat, ref_feat, grid_sample_index, grid_sample_xy):
    B, C, H, W = grid_feat.shape
    N = grid_sample_index.shape[1]
    BH = B // 2

    sc_fn = _build_sc_call(BH, C, H, W, N)
    halves = []
    for h in range(2):
        gf = grid_feat[h * BH:(h + 1) * BH]
        gsi = grid_sample_index[h * BH:(h + 1) * BH]
        tbl = _tc_pack_table(gf)
        idxh, w4 = _point_tables(gsi, BH, C, H, W, N)
        out3 = sc_fn(tbl, idxh, w4)
        halves.append(_tc_transpose(out3, (BH, C) + ref_feat.shape[2:]))
    return jnp.concatenate(halves, axis=0)


# speedup vs baseline: 1.8206x; 1.8206x over previous
import functools

import jax
import jax.numpy as jnp
from jax import lax
from jax.experimental import pallas as pl
from jax.experimental.pallas import tpu as pltpu, tpu_sc as plsc

_LANES = 16


def _build_sc_call(B, C, H, W, N):
    info = plsc.get_sparse_core_info()
    NC, NS = info.num_cores, info.num_subcores
    NW = NC * NS
    P = B * N
    PW = P // NW
    K = 128
    NCHUNK = PW // K
    WPB = NW // B
    CJ = C // _LANES
    NG = K // _LANES

    mesh = plsc.VectorSubcoreMesh(core_axis_name="c", subcore_axis_name="s")

    @functools.partial(
        pl.kernel,
        mesh=mesh,
        compiler_params=pltpu.CompilerParams(needs_layout_passes=False),
        out_type=jax.ShapeDtypeStruct((B, N, C), jnp.float32),
        scratch_types=[
            pltpu.VMEM((PW,), jnp.int32),
            pltpu.VMEM((PW,), jnp.int32),
            pltpu.VMEM((4, K), jnp.float32),
            pltpu.VMEM((4, K), jnp.float32),
            pltpu.VMEM((2 * K, 128), jnp.int32),
            pltpu.VMEM((2 * K, 128), jnp.int32),
            pltpu.VMEM((K, C), jnp.float32),
            pltpu.SemaphoreType.DMA,
            pltpu.SemaphoreType.DMA,
        ],
    )
    def sc_fn(tbl, i0h, i1h, w0h, w1h, w2h, w3h, out,
              idx0, idx1, w_a, w_b, rows_a, rows_b, outT, sem_a, sem_b):
        wid = lax.axis_index("s") * NC + lax.axis_index("c")
        bi = wid // WPB

        pltpu.sync_copy(i0h.at[pl.ds(wid * PW, PW)], idx0)
        pltpu.sync_copy(i1h.at[pl.ds(wid * PW, PW)], idx1)

        def fire(n, rows, wbuf, sem):
            pltpu.async_copy(tbl.at[idx0.at[pl.ds(n * K, K)]], rows.at[pl.ds(0, K)], sem)
            pltpu.async_copy(tbl.at[idx1.at[pl.ds(n * K, K)]], rows.at[pl.ds(K, K)], sem)
            for k, wh in enumerate((w0h, w1h, w2h, w3h)):
                pltpu.async_copy(wh.at[pl.ds(wid * PW + n * K, K)], wbuf.at[k], sem)

        def drain(rows, wbuf, sem):
            pltpu.make_async_copy(tbl.at[idx0.at[pl.ds(0, K)]], rows.at[pl.ds(0, K)], sem).wait()
            pltpu.make_async_copy(tbl.at[idx0.at[pl.ds(0, K)]], rows.at[pl.ds(K, K)], sem).wait()
            for k in range(4):
                pltpu.make_async_copy(w0h.at[pl.ds(0, K)], wbuf.at[k], sem).wait()

        hi = jnp.int32(-65536)

        def compute(rows, wbuf):
            def group_body(g, carry):
                gofs = g * _LANES
                wv0 = wbuf[0, pl.ds(gofs, _LANES)]
                wv1 = wbuf[1, pl.ds(gofs, _LANES)]
                wv2 = wbuf[2, pl.ds(gofs, _LANES)]
                wv3 = wbuf[3, pl.ds(gofs, _LANES)]
                for pp in range(_LANES):
                    p = gofs + pp
                    w0 = wv0[pp]
                    w1 = wv1[pp]
                    w2 = wv2[pp]
                    w3 = wv3[pp]
                    v0s = [rows[p, pl.ds(_LANES * g2, _LANES)] for g2 in range(CJ)]
                    v1s = [rows[K + p, pl.ds(_LANES * g2, _LANES)] for g2 in range(CJ)]
                    for g2 in range(CJ):
                        a = lax.bitcast_convert_type(v0s[g2] << 16, jnp.float32)
                        b = lax.bitcast_convert_type(v0s[g2] & hi, jnp.float32)
                        c = lax.bitcast_convert_type(v1s[g2] << 16, jnp.float32)
                        d = lax.bitcast_convert_type(v1s[g2] & hi, jnp.float32)
                        outT[p, pl.ds(_LANES * g2, _LANES)] = (w0 * a + w1 * b) + (w2 * c + w3 * d)
                return carry

            lax.fori_loop(0, NG, group_body, 0)

        def emit(n):
            offb = (wid % WPB) * PW + n * K
            pltpu.sync_copy(outT, out.at[bi, pl.ds(offb, K), :])

        fire(0, rows_a, w_a, sem_a)

        def pair_body(t, carry):
            n0 = 2 * t
            drain(rows_a, w_a, sem_a)
            fire(n0 + 1, rows_b, w_b, sem_b)
            compute(rows_a, w_a)
            emit(n0)
            drain(rows_b, w_b, sem_b)

            @pl.when(n0 + 2 < NCHUNK)
            def _():
                fire(n0 + 2, rows_a, w_a, sem_a)

            compute(rows_b, w_b)
            emit(n0 + 1)
            return carry

        lax.fori_loop(0, NCHUNK // 2, pair_body, 0)

    return sc_fn


def _tc_pack_table(grid_feat):
    B, C, H, W = grid_feat.shape
    HB = 8

    def body(x_ref, o_ref):
        xb = x_ref[...].astype(jnp.bfloat16)
        u = lax.bitcast_convert_type(xb, jnp.uint16).astype(jnp.uint32)
        un = jnp.concatenate([u[:, :, 1:], u[:, :, -1:]], axis=2)
        w = lax.bitcast_convert_type(u | (un << 16), jnp.int32)
        o_ref[:, 0:C] = w.reshape(C, HB * W).T
        o_ref[:, C:128] = jnp.zeros((HB * W, 128 - C), jnp.int32)

    return pl.pallas_call(
        body,
        grid=(B, H // HB),
        in_specs=[pl.BlockSpec((None, C, HB, W), lambda b, i: (b, 0, i, 0))],
        out_specs=pl.BlockSpec((HB * W, 128), lambda b, i: (b * (H // HB) + i, 0)),
        out_shape=jax.ShapeDtypeStruct((B * H * W, 128), jnp.int32),
    )(grid_feat)


def _tc_transpose(x, out_shape):
    B, N, C = x.shape
    _, _, P0, P1 = out_shape
    TN = 8192
    YS = TN // P1

    def body(x_ref, o_ref):
        for y in range(YS):
            o_ref[:, y, :] = x_ref[y * P1:(y + 1) * P1, :].T

    return pl.pallas_call(
        body,
        grid=(B, N // TN),
        in_specs=[pl.BlockSpec((None, TN, C), lambda b, i: (b, i, 0))],
        out_specs=pl.BlockSpec((None, C, YS, P1), lambda b, i: (b, 0, i, 0)),
        out_shape=jax.ShapeDtypeStruct(out_shape, x.dtype),
    )(x)


def kernel(grid_feat, ref_feat, grid_sample_index, grid_sample_xy):
    B, C, H, W = grid_feat.shape
    N = grid_sample_index.shape[1]
    P = B * N

    tbl = _tc_pack_table(grid_feat)

    gx = grid_sample_index[:B, :, 0, 0].reshape(P)
    gy = grid_sample_index[:B, :, 0, 1].reshape(P)
    x = (gx + 1.0) * 0.5 * (W - 1)
    y = (gy + 1.0) * 0.5 * (H - 1)
    x0 = jnp.floor(x)
    y0 = jnp.floor(y)
    wx1 = x - x0
    wx0 = 1.0 - wx1
    wy1 = y - y0
    wy0 = 1.0 - wy1
    xi0 = x0.astype(jnp.int32)
    yi0 = y0.astype(jnp.int32)
    bx = jnp.clip(xi0, 0, W - 2)
    by = jnp.clip(yi0, 0, H - 2)
    wxa = wx0 * (bx == xi0) + wx1 * (bx == xi0 + 1)
    wxb = wx0 * (bx + 1 == xi0) + wx1 * (bx + 1 == xi0 + 1)
    wya = wy0 * (by == yi0) + wy1 * (by == yi0 + 1)
    wyb = wy0 * (by + 1 == yi0) + wy1 * (by + 1 == yi0 + 1)

    bidx = jnp.repeat(jnp.arange(B, dtype=jnp.int32), N)
    idx0 = (bidx * H + by) * W + bx
    idx1 = idx0 + W

    sc_fn = _build_sc_call(B, C, H, W, N)
    out3 = sc_fn(tbl, idx0, idx1, wya * wxa, wya * wxb, wyb * wxa, wyb * wxb)
    return _tc_transpose(out3, ref_feat.shape)


# speedup vs baseline: 1.8236x; 1.0016x over previous
import functools

import jax
import jax.numpy as jnp
from jax import lax
from jax.experimental import pallas as pl
from jax.experimental.pallas import tpu as pltpu, tpu_sc as plsc

_LANES = 16


def _build_sc_call(B, C, H, W, N):
    info = plsc.get_sparse_core_info()
    NC, NS = info.num_cores, info.num_subcores
    NW = NC * NS
    P = B * N
    PW = P // NW
    K = 128
    NCHUNK = PW // K
    WPB = NW // B
    CJ = C // _LANES
    NG = K // _LANES

    mesh = plsc.VectorSubcoreMesh(core_axis_name="c", subcore_axis_name="s")

    @functools.partial(
        pl.kernel,
        mesh=mesh,
        compiler_params=pltpu.CompilerParams(needs_layout_passes=False),
        out_type=jax.ShapeDtypeStruct((B, N, C), jnp.float32),
        scratch_types=[
            pltpu.VMEM((PW,), jnp.int32),
            pltpu.VMEM((PW,), jnp.int32),
            pltpu.VMEM((4, K), jnp.float32),
            pltpu.VMEM((4, K), jnp.float32),
            pltpu.VMEM((2 * K, 128), jnp.int32),
            pltpu.VMEM((2 * K, 128), jnp.int32),
            pltpu.VMEM((K, C), jnp.float32),
            pltpu.VMEM((K, C), jnp.float32),
            pltpu.SemaphoreType.DMA,
            pltpu.SemaphoreType.DMA,
            pltpu.SemaphoreType.DMA,
            pltpu.SemaphoreType.DMA,
        ],
    )
    def sc_fn(tbl, i0h, i1h, w0h, w1h, w2h, w3h, out,
              idx0, idx1, w_a, w_b, rows_a, rows_b, outT_a, outT_b,
              sem_a, sem_b, sem_oa, sem_ob):
        wid = lax.axis_index("s") * NC + lax.axis_index("c")
        bi = wid // WPB

        pltpu.sync_copy(i0h.at[pl.ds(wid * PW, PW)], idx0)
        pltpu.sync_copy(i1h.at[pl.ds(wid * PW, PW)], idx1)

        def fire(n, rows, wbuf, sem):
            pltpu.async_copy(tbl.at[idx0.at[pl.ds(n * K, K)]], rows.at[pl.ds(0, K)], sem)
            pltpu.async_copy(tbl.at[idx1.at[pl.ds(n * K, K)]], rows.at[pl.ds(K, K)], sem)
            for k, wh in enumerate((w0h, w1h, w2h, w3h)):
                pltpu.async_copy(wh.at[pl.ds(wid * PW + n * K, K)], wbuf.at[k], sem)

        def drain(rows, wbuf, sem):
            pltpu.make_async_copy(tbl.at[idx0.at[pl.ds(0, K)]], rows.at[pl.ds(0, K)], sem).wait()
            pltpu.make_async_copy(tbl.at[idx0.at[pl.ds(0, K)]], rows.at[pl.ds(K, K)], sem).wait()
            for k in range(4):
                pltpu.make_async_copy(w0h.at[pl.ds(0, K)], wbuf.at[k], sem).wait()

        hi = jnp.int32(-65536)

        def compute(rows, wbuf, outT):
            def group_body(g, carry):
                gofs = g * _LANES
                wv0 = wbuf[0, pl.ds(gofs, _LANES)]
                wv1 = wbuf[1, pl.ds(gofs, _LANES)]
                wv2 = wbuf[2, pl.ds(gofs, _LANES)]
                wv3 = wbuf[3, pl.ds(gofs, _LANES)]
                for pp in range(_LANES):
                    p = gofs + pp
                    w0 = wv0[pp]
                    w1 = wv1[pp]
                    w2 = wv2[pp]
                    w3 = wv3[pp]
                    v0s = [rows[p, pl.ds(_LANES * g2, _LANES)] for g2 in range(CJ)]
                    v1s = [rows[K + p, pl.ds(_LANES * g2, _LANES)] for g2 in range(CJ)]
                    for g2 in range(CJ):
                        a = lax.bitcast_convert_type(v0s[g2] << 16, jnp.float32)
                        b = lax.bitcast_convert_type(v0s[g2] & hi, jnp.float32)
                        c = lax.bitcast_convert_type(v1s[g2] << 16, jnp.float32)
                        d = lax.bitcast_convert_type(v1s[g2] & hi, jnp.float32)
                        outT[p, pl.ds(_LANES * g2, _LANES)] = (w0 * a + w1 * b) + (w2 * c + w3 * d)
                return carry

            lax.fori_loop(0, NG, group_body, 0)

        def emit(n, outT, sem):
            offb = (wid % WPB) * PW + n * K
            pltpu.async_copy(outT, out.at[bi, pl.ds(offb, K), :], sem)

        def emit_wait(outT, sem):
            pltpu.make_async_copy(outT, out.at[bi, pl.ds(0, K), :], sem).wait()

        fire(0, rows_a, w_a, sem_a)

        def pair_body(t, carry):
            n0 = 2 * t
            drain(rows_a, w_a, sem_a)
            fire(n0 + 1, rows_b, w_b, sem_b)

            @pl.when(t > 0)
            def _():
                emit_wait(outT_a, sem_oa)

            compute(rows_a, w_a, outT_a)
            emit(n0, outT_a, sem_oa)
            drain(rows_b, w_b, sem_b)

            @pl.when(n0 + 2 < NCHUNK)
            def _():
                fire(n0 + 2, rows_a, w_a, sem_a)

            @pl.when(t > 0)
            def _():
                emit_wait(outT_b, sem_ob)

            compute(rows_b, w_b, outT_b)
            emit(n0 + 1, outT_b, sem_ob)
            return carry

        lax.fori_loop(0, NCHUNK // 2, pair_body, 0)
        emit_wait(outT_a, sem_oa)
        emit_wait(outT_b, sem_ob)

    return sc_fn


def _tc_pack_table(grid_feat):
    B, C, H, W = grid_feat.shape
    HB = 8

    def body(x_ref, o_ref):
        xb = x_ref[...].astype(jnp.bfloat16)
        u = lax.bitcast_convert_type(xb, jnp.uint16).astype(jnp.uint32)
        un = jnp.concatenate([u[:, :, 1:], u[:, :, -1:]], axis=2)
        w = lax.bitcast_convert_type(u | (un << 16), jnp.int32)
        o_ref[:, 0:C] = w.reshape(C, HB * W).T

    return pl.pallas_call(
        body,
        grid=(B, H // HB),
        in_specs=[pl.BlockSpec((None, C, HB, W), lambda b, i: (b, 0, i, 0))],
        out_specs=pl.BlockSpec((HB * W, 128), lambda b, i: (b * (H // HB) + i, 0)),
        out_shape=jax.ShapeDtypeStruct((B * H * W, 128), jnp.int32),
    )(grid_feat)


def _tc_transpose(x, out_shape):
    B, N, C = x.shape
    _, _, P0, P1 = out_shape
    TN = 8192
    YS = TN // P1

    def body(x_ref, o_ref):
        for y in range(YS):
            o_ref[:, y, :] = x_ref[y * P1:(y + 1) * P1, :].T

    return pl.pallas_call(
        body,
        grid=(B, N // TN),
        in_specs=[pl.BlockSpec((None, TN, C), lambda b, i: (b, i, 0))],
        out_specs=pl.BlockSpec((None, C, YS, P1), lambda b, i: (b, 0, i, 0)),
        out_shape=jax.ShapeDtypeStruct(out_shape, x.dtype),
    )(x)


def kernel(grid_feat, ref_feat, grid_sample_index, grid_sample_xy):
    B, C, H, W = grid_feat.shape
    N = grid_sample_index.shape[1]
    P = B * N

    tbl = _tc_pack_table(grid_feat)

    gx = grid_sample_index[:B, :, 0, 0].reshape(P)
    gy = grid_sample_index[:B, :, 0, 1].reshape(P)
    x = (gx + 1.0) * 0.5 * (W - 1)
    y = (gy + 1.0) * 0.5 * (H - 1)
    x0 = jnp.floor(x)
    y0 = jnp.floor(y)
    wx1 = x - x0
    wx0 = 1.0 - wx1
    wy1 = y - y0
    wy0 = 1.0 - wy1
    xi0 = x0.astype(jnp.int32)
    yi0 = y0.astype(jnp.int32)
    bx = jnp.clip(xi0, 0, W - 2)
    by = jnp.clip(yi0, 0, H - 2)
    wxa = wx0 * (bx == xi0) + wx1 * (bx == xi0 + 1)
    wxb = wx0 * (bx + 1 == xi0) + wx1 * (bx + 1 == xi0 + 1)
    wya = wy0 * (by == yi0) + wy1 * (by == yi0 + 1)
    wyb = wy0 * (by + 1 == yi0) + wy1 * (by + 1 == yi0 + 1)

    bidx = jnp.repeat(jnp.arange(B, dtype=jnp.int32), N)
    idx0 = (bidx * H + by) * W + bx
    idx1 = idx0 + W

    sc_fn = _build_sc_call(B, C, H, W, N)
    out3 = sc_fn(tbl, idx0, idx1, wya * wxa, wya * wxb, wyb * wxa, wyb * wxb)
    return _tc_transpose(out3, ref_feat.shape)


# speedup vs baseline: 1.9568x; 1.0731x over previous
import functools

import jax
import jax.numpy as jnp
from jax import lax
from jax.experimental import pallas as pl
from jax.experimental.pallas import tpu as pltpu, tpu_sc as plsc

_LANES = 16


def _build_sc_call(B, C, H, W, N):
    info = plsc.get_sparse_core_info()
    NC, NS = info.num_cores, info.num_subcores
    NW = NC * NS
    P = B * N
    PW = P // NW
    K = 128
    NCHUNK = PW // K
    WPB = NW // B
    CJ = C // _LANES
    NG = K // _LANES

    mesh = plsc.VectorSubcoreMesh(core_axis_name="c", subcore_axis_name="s")

    @functools.partial(
        pl.kernel,
        mesh=mesh,
        compiler_params=pltpu.CompilerParams(needs_layout_passes=False),
        out_type=jax.ShapeDtypeStruct((B, N, C), jnp.float32),
        scratch_types=[
            pltpu.VMEM((PW,), jnp.int32),
            pltpu.VMEM((PW,), jnp.int32),
            pltpu.VMEM((4, K), jnp.float32),
            pltpu.VMEM((4, K), jnp.float32),
            pltpu.VMEM((2 * K, 128), jnp.int32),
            pltpu.VMEM((2 * K, 128), jnp.int32),
            pltpu.VMEM((K, C), jnp.float32),
            pltpu.VMEM((K, C), jnp.float32),
            pltpu.SemaphoreType.DMA,
            pltpu.SemaphoreType.DMA,
            pltpu.SemaphoreType.DMA,
            pltpu.SemaphoreType.DMA,
        ],
    )
    def sc_fn(tbl, i0h, i1h, w0h, w1h, w2h, w3h, out,
              idx0, idx1, w_a, w_b, rows_a, rows_b, outT_a, outT_b,
              sem_a, sem_b, sem_oa, sem_ob):
        wid = lax.axis_index("s") * NC + lax.axis_index("c")
        bi = wid // WPB

        pltpu.sync_copy(i0h.at[pl.ds(wid * PW, PW)], idx0)
        pltpu.sync_copy(i1h.at[pl.ds(wid * PW, PW)], idx1)

        def fire(n, rows, wbuf, sem):
            pltpu.async_copy(tbl.at[idx0.at[pl.ds(n * K, K)]], rows.at[pl.ds(0, K)], sem)
            pltpu.async_copy(tbl.at[idx1.at[pl.ds(n * K, K)]], rows.at[pl.ds(K, K)], sem)
            for k, wh in enumerate((w0h, w1h, w2h, w3h)):
                pltpu.async_copy(wh.at[pl.ds(wid * PW + n * K, K)], wbuf.at[k], sem)

        def drain(rows, wbuf, sem):
            pltpu.make_async_copy(tbl.at[idx0.at[pl.ds(0, K)]], rows.at[pl.ds(0, K)], sem).wait()
            pltpu.make_async_copy(tbl.at[idx0.at[pl.ds(0, K)]], rows.at[pl.ds(K, K)], sem).wait()
            for k in range(4):
                pltpu.make_async_copy(w0h.at[pl.ds(0, K)], wbuf.at[k], sem).wait()

        hi = jnp.int32(-65536)

        def compute(rows, wbuf, outT):
            def group_body(g, carry):
                gofs = g * _LANES
                wv0 = wbuf[0, pl.ds(gofs, _LANES)]
                wv1 = wbuf[1, pl.ds(gofs, _LANES)]
                wv2 = wbuf[2, pl.ds(gofs, _LANES)]
                wv3 = wbuf[3, pl.ds(gofs, _LANES)]
                for pp in range(_LANES):
                    p = gofs + pp
                    w0 = wv0[pp]
                    w1 = wv1[pp]
                    w2 = wv2[pp]
                    w3 = wv3[pp]
                    v0s = [rows[p, pl.ds(_LANES * g2, _LANES)] for g2 in range(CJ)]
                    v1s = [rows[K + p, pl.ds(_LANES * g2, _LANES)] for g2 in range(CJ)]
                    for g2 in range(CJ):
                        a = lax.bitcast_convert_type(v0s[g2] << 16, jnp.float32)
                        b = lax.bitcast_convert_type(v0s[g2] & hi, jnp.float32)
                        c = lax.bitcast_convert_type(v1s[g2] << 16, jnp.float32)
                        d = lax.bitcast_convert_type(v1s[g2] & hi, jnp.float32)
                        outT[p, pl.ds(_LANES * g2, _LANES)] = (w0 * a + w1 * b) + (w2 * c + w3 * d)
                return carry

            lax.fori_loop(0, NG, group_body, 0)

        def emit(n, outT, sem):
            offb = (wid % WPB) * PW + n * K
            pltpu.async_copy(outT, out.at[bi, pl.ds(offb, K), :], sem)

        def emit_wait(outT, sem):
            pltpu.make_async_copy(outT, out.at[bi, pl.ds(0, K), :], sem).wait()

        fire(0, rows_a, w_a, sem_a)

        def pair_body(t, carry):
            n0 = 2 * t
            drain(rows_a, w_a, sem_a)
            fire(n0 + 1, rows_b, w_b, sem_b)

            @pl.when(t > 0)
            def _():
                emit_wait(outT_a, sem_oa)

            compute(rows_a, w_a, outT_a)
            emit(n0, outT_a, sem_oa)
            drain(rows_b, w_b, sem_b)

            @pl.when(n0 + 2 < NCHUNK)
            def _():
                fire(n0 + 2, rows_a, w_a, sem_a)

            @pl.when(t > 0)
            def _():
                emit_wait(outT_b, sem_ob)

            compute(rows_b, w_b, outT_b)
            emit(n0 + 1, outT_b, sem_ob)
            return carry

        lax.fori_loop(0, NCHUNK // 2, pair_body, 0)
        emit_wait(outT_a, sem_oa)
        emit_wait(outT_b, sem_ob)

    return sc_fn


def _tc_pack_table(grid_feat):
    B, C, H, W = grid_feat.shape
    HB = 16

    def body(x_ref, o_ref):
        xb = x_ref[...].astype(jnp.bfloat16)
        u = lax.bitcast_convert_type(xb, jnp.uint16).astype(jnp.uint32)
        un = jnp.concatenate([u[:, :, 1:], u[:, :, -1:]], axis=2)
        w = lax.bitcast_convert_type(u | (un << 16), jnp.int32)
        o_ref[:, 0:C] = w.reshape(C, HB * W).T

    return pl.pallas_call(
        body,
        grid=(B, H // HB),
        in_specs=[pl.BlockSpec((None, C, HB, W), lambda b, i: (b, 0, i, 0))],
        out_specs=pl.BlockSpec((HB * W, 128), lambda b, i: (b * (H // HB) + i, 0)),
        out_shape=jax.ShapeDtypeStruct((B * H * W, 128), jnp.int32),
    )(grid_feat)


def _tc_transpose(x, out_shape):
    B, N, C = x.shape
    _, _, P0, P1 = out_shape
    TN = 8192
    YS = TN // P1

    def body(x_ref, o_ref):
        o_ref[...] = x_ref[...].T.reshape(C, YS, P1)

    return pl.pallas_call(
        body,
        grid=(B, N // TN),
        in_specs=[pl.BlockSpec((None, TN, C), lambda b, i: (b, i, 0))],
        out_specs=pl.BlockSpec((None, C, YS, P1), lambda b, i: (b, 0, i, 0)),
        out_shape=jax.ShapeDtypeStruct(out_shape, x.dtype),
    )(x)


def kernel(grid_feat, ref_feat, grid_sample_index, grid_sample_xy):
    B, C, H, W = grid_feat.shape
    N = grid_sample_index.shape[1]
    P = B * N

    tbl = _tc_pack_table(grid_feat)

    gx = grid_sample_index[:B, :, 0, 0].reshape(P)
    gy = grid_sample_index[:B, :, 0, 1].reshape(P)
    x = (gx + 1.0) * 0.5 * (W - 1)
    y = (gy + 1.0) * 0.5 * (H - 1)
    x0 = jnp.floor(x)
    y0 = jnp.floor(y)
    wx1 = x - x0
    wx0 = 1.0 - wx1
    wy1 = y - y0
    wy0 = 1.0 - wy1
    xi0 = x0.astype(jnp.int32)
    yi0 = y0.astype(jnp.int32)
    bx = jnp.clip(xi0, 0, W - 2)
    by = jnp.clip(yi0, 0, H - 2)
    wxa = wx0 * (bx == xi0) + wx1 * (bx == xi0 + 1)
    wxb = wx0 * (bx + 1 == xi0) + wx1 * (bx + 1 == xi0 + 1)
    wya = wy0 * (by == yi0) + wy1 * (by == yi0 + 1)
    wyb = wy0 * (by + 1 == yi0) + wy1 * (by + 1 == yi0 + 1)

    bidx = jnp.repeat(jnp.arange(B, dtype=jnp.int32), N)
    idx0 = (bidx * H + by) * W + bx
    idx1 = idx0 + W

    sc_fn = _build_sc_call(B, C, H, W, N)
    out3 = sc_fn(tbl, idx0, idx1, wya * wxa, wya * wxb, wyb * wxa, wyb * wxb)
    return _tc_transpose(out3, ref_feat.shape)


# speedup vs baseline: 1.9862x; 1.0151x over previous
import functools

import jax
import jax.numpy as jnp
from jax import lax
from jax.experimental import pallas as pl
from jax.experimental.pallas import tpu as pltpu, tpu_sc as plsc

_LANES = 16


def _build_sc_call(B, C, H, W, N):
    info = plsc.get_sparse_core_info()
    NC, NS = info.num_cores, info.num_subcores
    NW = NC * NS
    P = B * N
    PW = P // NW
    K = 128
    NCHUNK = PW // K
    WPB = NW // B
    CJ = C // _LANES
    NG = K // _LANES

    mesh = plsc.VectorSubcoreMesh(core_axis_name="c", subcore_axis_name="s")

    @functools.partial(
        pl.kernel,
        mesh=mesh,
        compiler_params=pltpu.CompilerParams(needs_layout_passes=False),
        out_type=jax.ShapeDtypeStruct((B, N, C), jnp.float32),
        scratch_types=[
            pltpu.VMEM((PW,), jnp.int32),
            pltpu.VMEM((PW,), jnp.int32),
            pltpu.VMEM((4, K), jnp.float32),
            pltpu.VMEM((4, K), jnp.float32),
            pltpu.VMEM((2 * K, 128), jnp.int32),
            pltpu.VMEM((2 * K, 128), jnp.int32),
            pltpu.VMEM((K, C), jnp.float32),
            pltpu.VMEM((K, C), jnp.float32),
            pltpu.SemaphoreType.DMA,
            pltpu.SemaphoreType.DMA,
            pltpu.SemaphoreType.DMA,
            pltpu.SemaphoreType.DMA,
        ],
    )
    def sc_fn(tbl, i0h, i1h, w0h, w1h, w2h, w3h, out,
              idx0, idx1, w_a, w_b, rows_a, rows_b, outT_a, outT_b,
              sem_a, sem_b, sem_oa, sem_ob):
        wid = lax.axis_index("s") * NC + lax.axis_index("c")
        bi = wid // WPB

        pltpu.sync_copy(i0h.at[pl.ds(wid * PW, PW)], idx0)
        pltpu.sync_copy(i1h.at[pl.ds(wid * PW, PW)], idx1)

        def fire(n, rows, wbuf, sem):
            pltpu.async_copy(tbl.at[idx0.at[pl.ds(n * K, K)]], rows.at[pl.ds(0, K)], sem)
            pltpu.async_copy(tbl.at[idx1.at[pl.ds(n * K, K)]], rows.at[pl.ds(K, K)], sem)
            for k, wh in enumerate((w0h, w1h, w2h, w3h)):
                pltpu.async_copy(wh.at[pl.ds(wid * PW + n * K, K)], wbuf.at[k], sem)

        def drain(rows, wbuf, sem):
            pltpu.make_async_copy(tbl.at[idx0.at[pl.ds(0, K)]], rows.at[pl.ds(0, K)], sem).wait()
            pltpu.make_async_copy(tbl.at[idx0.at[pl.ds(0, K)]], rows.at[pl.ds(K, K)], sem).wait()
            for k in range(4):
                pltpu.make_async_copy(w0h.at[pl.ds(0, K)], wbuf.at[k], sem).wait()

        hi = jnp.int32(-65536)

        def compute(rows, wbuf, outT):
            def group_body(g, carry):
                gofs = g * _LANES
                wv0 = wbuf[0, pl.ds(gofs, _LANES)]
                wv1 = wbuf[1, pl.ds(gofs, _LANES)]
                wv2 = wbuf[2, pl.ds(gofs, _LANES)]
                wv3 = wbuf[3, pl.ds(gofs, _LANES)]
                for pp in range(_LANES):
                    p = gofs + pp
                    w0 = wv0[pp]
                    w1 = wv1[pp]
                    w2 = wv2[pp]
                    w3 = wv3[pp]
                    v0s = [rows[p, pl.ds(_LANES * g2, _LANES)] for g2 in range(CJ)]
                    v1s = [rows[K + p, pl.ds(_LANES * g2, _LANES)] for g2 in range(CJ)]
                    for g2 in range(CJ):
                        a = lax.bitcast_convert_type(v0s[g2] << 16, jnp.float32)
                        b = lax.bitcast_convert_type(v0s[g2] & hi, jnp.float32)
                        c = lax.bitcast_convert_type(v1s[g2] << 16, jnp.float32)
                        d = lax.bitcast_convert_type(v1s[g2] & hi, jnp.float32)
                        outT[p, pl.ds(_LANES * g2, _LANES)] = (w0 * a + w1 * b) + (w2 * c + w3 * d)
                return carry

            lax.fori_loop(0, NG, group_body, 0)

        def emit(n, outT, sem):
            offb = (wid % WPB) * PW + n * K
            pltpu.async_copy(outT, out.at[bi, pl.ds(offb, K), :], sem)

        def emit_wait(outT, sem):
            pltpu.make_async_copy(outT, out.at[bi, pl.ds(0, K), :], sem).wait()

        fire(0, rows_a, w_a, sem_a)

        def pair_body(t, carry):
            n0 = 2 * t
            drain(rows_a, w_a, sem_a)
            fire(n0 + 1, rows_b, w_b, sem_b)

            @pl.when(t > 0)
            def _():
                emit_wait(outT_a, sem_oa)

            compute(rows_a, w_a, outT_a)
            emit(n0, outT_a, sem_oa)
            drain(rows_b, w_b, sem_b)

            @pl.when(n0 + 2 < NCHUNK)
            def _():
                fire(n0 + 2, rows_a, w_a, sem_a)

            @pl.when(t > 0)
            def _():
                emit_wait(outT_b, sem_ob)

            compute(rows_b, w_b, outT_b)
            emit(n0 + 1, outT_b, sem_ob)
            return carry

        lax.fori_loop(0, NCHUNK // 2, pair_body, 0)
        emit_wait(outT_a, sem_oa)
        emit_wait(outT_b, sem_ob)

    return sc_fn


def _tc_pack_table(grid_feat):
    B, C, H, W = grid_feat.shape
    HB = 16

    def body(x_ref, o_ref):
        xb = x_ref[...].astype(jnp.bfloat16)
        u = lax.bitcast_convert_type(xb, jnp.uint16).astype(jnp.uint32)
        un = jnp.concatenate([u[:, :, 1:], u[:, :, -1:]], axis=2)
        w = lax.bitcast_convert_type(u | (un << 16), jnp.int32)
        o_ref[:, 0:C] = w.reshape(C, HB * W).T

    return pl.pallas_call(
        body,
        grid=(B, H // HB),
        in_specs=[pl.BlockSpec((None, C, HB, W), lambda b, i: (b, 0, i, 0))],
        out_specs=pl.BlockSpec((HB * W, 128), lambda b, i: (b * (H // HB) + i, 0)),
        out_shape=jax.ShapeDtypeStruct((B * H * W, 128), jnp.int32),
    )(grid_feat)


def _tc_transpose(x, out_shape):
    B, N, C = x.shape
    _, _, P0, P1 = out_shape
    TN = 16384
    YS = TN // P1

    def body(x_ref, o_ref):
        o_ref[...] = x_ref[...].T.reshape(C, YS, P1)

    return pl.pallas_call(
        body,
        grid=(B, N // TN),
        in_specs=[pl.BlockSpec((None, TN, C), lambda b, i: (b, i, 0))],
        out_specs=pl.BlockSpec((None, C, YS, P1), lambda b, i: (b, 0, i, 0)),
        out_shape=jax.ShapeDtypeStruct(out_shape, x.dtype),
    )(x)


def kernel(grid_feat, ref_feat, grid_sample_index, grid_sample_xy):
    B, C, H, W = grid_feat.shape
    N = grid_sample_index.shape[1]
    P = B * N

    tbl = _tc_pack_table(grid_feat)

    gx = grid_sample_index[:B, :, 0, 0].reshape(P)
    gy = grid_sample_index[:B, :, 0, 1].reshape(P)
    x = (gx + 1.0) * 0.5 * (W - 1)
    y = (gy + 1.0) * 0.5 * (H - 1)
    x0 = jnp.floor(x)
    y0 = jnp.floor(y)
    wx1 = x - x0
    wx0 = 1.0 - wx1
    wy1 = y - y0
    wy0 = 1.0 - wy1
    xi0 = x0.astype(jnp.int32)
    yi0 = y0.astype(jnp.int32)
    bx = jnp.clip(xi0, 0, W - 2)
    by = jnp.clip(yi0, 0, H - 2)
    wxa = wx0 * (bx == xi0) + wx1 * (bx == xi0 + 1)
    wxb = wx0 * (bx + 1 == xi0) + wx1 * (bx + 1 == xi0 + 1)
    wya = wy0 * (by == yi0) + wy1 * (by == yi0 + 1)
    wyb = wy0 * (by + 1 == yi0) + wy1 * (by + 1 == yi0 + 1)

    bidx = jnp.repeat(jnp.arange(B, dtype=jnp.int32), N)
    idx0 = (bidx * H + by) * W + bx
    idx1 = idx0 + W

    sc_fn = _build_sc_call(B, C, H, W, N)
    out3 = sc_fn(tbl, idx0, idx1, wya * wxa, wya * wxb, wyb * wxa, wyb * wxb)
    return _tc_transpose(out3, ref_feat.shape)
